# baseline (speedup 1.0000x reference)
import functools

import jax
import jax.numpy as jnp
from jax import lax
from jax.experimental import pallas as pl
from jax.experimental.pallas import tpu as pltpu
from jax.experimental.pallas import tpu_sc as plsc

B, U, T = 4096, 2048, 1024
NC, NS, L = 2, 16, 16
NW = NC * NS
RPW = B // NW
NBIN = T + L
BR = 8
NBLK = RPW // BR
BU = BR * U


def kernel(x, w, b):
    w_flat = w.reshape(-1)
    mesh = plsc.VectorSubcoreMesh(core_axis_name="c", subcore_axis_name="s")

    @functools.partial(
        pl.kernel,
        mesh=mesh,
        out_type=jax.ShapeDtypeStruct((B, T), jnp.float32),
        compiler_params=pltpu.CompilerParams(
            needs_layout_passes=False, use_tc_tiling_on_sc=False),
        scratch_types=[
            pltpu.VMEM((BR, U), jnp.int32),
            pltpu.VMEM((BR, U), jnp.int32),
            pltpu.VMEM((BU,), jnp.int32),
            pltpu.VMEM((BU,), jnp.int32),
            pltpu.VMEM((BU,), jnp.float32),
            pltpu.VMEM((BU,), jnp.float32),
            pltpu.VMEM((BR, NBIN), jnp.float32),
            pltpu.VMEM((BR, NBIN), jnp.float32),
            pltpu.VMEM((T,), jnp.float32),
            pltpu.SemaphoreType.DMA,
            pltpu.SemaphoreType.DMA,
            pltpu.SemaphoreType.DMA,
            pltpu.SemaphoreType.DMA,
            pltpu.SemaphoreType.DMA,
            pltpu.SemaphoreType.DMA,
        ],
    )
    def sck(x_hbm, w_hbm, b_hbm, out_hbm,
            xb0, xb1, gib0, gib1, gvb0, gvb1, accb0, accb1, bias,
            sx0, sx1, sg0, sg1, so0, so1):
        xb = (xb0, xb1)
        gib = (gib0, gib1)
        gvb = (gvb0, gvb1)
        accb = (accb0, accb1)
        sx = (sx0, sx1)
        sg = (sg0, sg1)
        so = (so0, so1)
        wid = lax.axis_index("s") * NC + lax.axis_index("c")
        row0 = wid * RPW
        pltpu.sync_copy(b_hbm, bias)
        col = lax.iota(jnp.int32, L)

        def xblk(k):
            return x_hbm.at[pl.ds(row0 + k * BR, BR), :]

        def oblk(k):
            return out_hbm.at[pl.ds(row0 + k * BR, BR), :]

        def compute_idx(xr, gi):
            for r in range(BR):
                @plsc.parallel_loop(0, U // L, unroll=8)
                def _(i, r=r):
                    xv = xr[r, pl.ds(i * L, L)]
                    flat = xv * U + (col + (i * L - U))
                    gi[pl.ds(r * U + i * L, L)] = jnp.maximum(flat, 0)

        def init_acc(a):
            for r in range(BR):
                @plsc.parallel_loop(0, T // L, unroll=8)
                def _(j, r=r):
                    a[r, pl.ds(j * L + L, L)] = bias[pl.ds(j * L, L)]

        def scatter_blk(xr, gv, a):
            for r in range(BR):
                rowv = jnp.full((L,), r, jnp.int32)

                def si(i, c, r=r, rowv=rowv):
                    xv = xr[r, pl.ds(i * L, L)]
                    vv = gv[pl.ds(r * U + i * L, L)]
                    plsc.addupdate_scatter(a, [rowv, xv + (L - 1)], vv)
                    return c
                lax.fori_loop(0, U // L, si, 0, unroll=8)

        def handle(k, p, first_pair):
            q = 1 - p
            pltpu.make_async_copy(xblk(k), xb[p], sx[p]).wait()
            compute_idx(xb[p], gib[p])
            pltpu.async_copy(w_hbm.at[gib[p]], gvb[p], sg[p])
            if not first_pair:
                pltpu.make_async_copy(
                    accb[p].at[:, pl.ds(L, T)], oblk(k), so[p]).wait()
            init_acc(accb[p])
            pltpu.make_async_copy(w_hbm.at[gib[q]], gvb[q], sg[q]).wait()
            scatter_blk(xb[q], gvb[q], accb[q])
            pltpu.async_copy(accb[q].at[:, pl.ds(L, T)], oblk(k - 1), so[q])
            nxt = jnp.minimum(k + 1, NBLK - 1)
            pltpu.async_copy(xblk(nxt), xb[q], sx[q])

        pltpu.sync_copy(xblk(0), xb0)
        compute_idx(xb0, gib0)
        pltpu.async_copy(w_hbm.at[gib0], gvb0, sg0)
        pltpu.async_copy(xblk(1), xb1, sx1)
        init_acc(accb0)
        handle(1, 1, True)

        def pair_body(j, c):
            handle(2 * j, 0, False)
            handle(2 * j + 1, 1, False)
            return c
        lax.fori_loop(1, NBLK // 2, pair_body, 0)

        pltpu.make_async_copy(w_hbm.at[gib1], gvb1, sg1).wait()
        scatter_blk(xb1, gvb1, accb1)
        pltpu.sync_copy(accb1.at[:, pl.ds(L, T)], oblk(NBLK - 1))
        pltpu.make_async_copy(
            accb0.at[:, pl.ds(L, T)], oblk(NBLK - 1), so0).wait()
        pltpu.make_async_copy(xblk(NBLK - 1), xb0, sx0).wait()

    return sck(x, w_flat, b)
